# baseline (speedup 1.0000x reference)
import functools

import jax
import jax.numpy as jnp
from jax import lax
from jax.experimental import pallas as pl
from jax.experimental.pallas import tpu as pltpu
from jax.experimental.pallas import tpu_sc as plsc

_N = 100000
_E = 1600000
_NG = 128
_D = 64
_NB = 4
_R = 25088
_NPAD = _NB * _R
_RS = _R // 16
_CH = 128
_SUB = 6
_ET = 100608
_NSUP = _ET // (_SUB * _CH)
_EPAD = 16 * _ET
_EROWS = _EPAD // _CH
_ETD = _EPAD // 32
_NCHD = _ETD // _CH
_DS = 6288
_BM = 512
_NBLK = _NPAD // _BM


def _make_deg():
    mesh = plsc.VectorSubcoreMesh(core_axis_name="c", subcore_axis_name="s")

    @functools.partial(
        pl.kernel,
        out_type=jax.ShapeDtypeStruct((2, _NPAD), jnp.float32),
        mesh=mesh,
        compiler_params=pltpu.CompilerParams(use_tc_tiling_on_sc=False),
        scratch_types=[
            pltpu.VMEM_SHARED((16 * _DS,), jnp.float32),
            pltpu.VMEM((_CH,), jnp.int32),
            pltpu.VMEM((_CH,), jnp.float32),
            pltpu.VMEM((_DS,), jnp.float32),
        ],
    )
    def deg(dstp_hbm, out_hbm, acc, didx, ones, zbuf):
        c = lax.axis_index("c")
        s = lax.axis_index("s")
        one_v = jnp.full((16,), 1.0, jnp.float32)
        zero_v = jnp.zeros((16,), jnp.float32)
        for j in range(_CH // 16):
            ones[pl.ds(j * 16, 16)] = one_v

        def zb(i, carry):
            zbuf[pl.ds(i * 16, 16)] = zero_v
            return carry

        lax.fori_loop(0, _DS // 16, zb, 0)
        pltpu.sync_copy(zbuf, acc.at[pl.ds(s * _DS, _DS)])
        plsc.subcore_barrier()

        base_r = (c * 16 + s) * _NCHD

        def chunk(g, carry):
            pltpu.sync_copy(dstp_hbm.at[base_r + g], didx)
            pltpu.sync_copy(ones, acc.at[didx], add=True)
            return carry

        lax.fori_loop(0, _NCHD, chunk, 0)
        plsc.subcore_barrier()
        pltpu.sync_copy(acc.at[pl.ds(s * (_NPAD // 16), _NPAD // 16)],
                        out_hbm.at[c, pl.ds(s * (_NPAD // 16), _NPAD // 16)])

    return deg


def _make_propagate():
    mesh = plsc.VectorSubcoreMesh(core_axis_name="c", subcore_axis_name="s")

    @functools.partial(
        pl.kernel,
        out_type=jax.ShapeDtypeStruct((_NPAD, _D), jnp.float32),
        mesh=mesh,
        compiler_params=pltpu.CompilerParams(use_tc_tiling_on_sc=False),
        scratch_types=[
            pltpu.VMEM_SHARED((_R + 1, _D), jnp.float32),
            pltpu.VMEM((3, _SUB, _CH), jnp.int32),
            pltpu.VMEM((3, _SUB, _CH), jnp.int32),
            pltpu.VMEM((3, _CH, _D), jnp.float32),
            pltpu.SemaphoreType.DMA((3,)),
            pltpu.SemaphoreType.DMA((3,)),
            pltpu.SemaphoreType.DMA((2,)),
        ],
    )
    def prop(t_hbm, srcp_hbm, dstp_hbm, out_hbm, acc, sidx, didx, rows,
             gsem, ssem, isem):
        c = lax.axis_index("c")
        s = lax.axis_index("s")
        tile_r0 = s * (_ET // _CH)

        def idx_load(g, ph):
            roff = tile_r0 + g * _SUB
            pltpu.async_copy(srcp_hbm.at[pl.ds(roff, _SUB)], sidx.at[ph],
                             isem.at[0])
            pltpu.async_copy(dstp_hbm.at[pl.ds(roff, _SUB)], didx.at[ph],
                             isem.at[1])

        def idx_wait(g, ph):
            roff = tile_r0 + g * _SUB
            pltpu.make_async_copy(srcp_hbm.at[pl.ds(roff, _SUB)], sidx.at[ph],
                                  isem.at[0]).wait()
            pltpu.make_async_copy(dstp_hbm.at[pl.ds(roff, _SUB)], didx.at[ph],
                                  isem.at[1]).wait()

        def s_wait(ph, r):
            pltpu.make_async_copy(rows.at[r], acc.at[didx.at[ph, 0]],
                                  ssem.at[r]).wait()

        def g_wait(ph, j):
            pltpu.make_async_copy(t_hbm.at[sidx.at[ph, j]], rows.at[j % 3],
                                  gsem.at[j % 3]).wait()

        def s_fire(ph, j):
            pltpu.async_copy(rows.at[j % 3], acc.at[didx.at[ph, j]],
                             ssem.at[j % 3], add=True)

        for bb in range(2):
            b = 2 * c + bb
            base = b * _R
            pltpu.sync_copy(t_hbm.at[pl.ds(base + s * _RS, _RS)],
                            acc.at[pl.ds(s * _RS, _RS)])
            idx_load(0, 0)
            plsc.subcore_barrier()

            def sup(g, carry):
                ph = lax.rem(g, 3)
                pph = lax.rem(g + 2, 3)
                idx_wait(g, ph)

                @pl.when(g + 1 < _NSUP)
                def _():
                    idx_load(g + 1, lax.rem(g + 1, 3))

                for j in range(_SUB):
                    for k in range(_CH // 16):
                        dv = didx[ph, j, pl.ds(k * 16, 16)] - base
                        ok = (dv >= 0) & (dv < _R)
                        didx[ph, j, pl.ds(k * 16, 16)] = jnp.where(ok, dv, _R)
                for j in range(_SUB):
                    r = j % 3
                    if j < 3:
                        @pl.when(g > 0)
                        def _(ph=ph, r=r):
                            s_wait(ph, r)
                    else:
                        s_wait(ph, r)
                    pltpu.async_copy(t_hbm.at[sidx.at[ph, j]],
                                     rows.at[r], gsem.at[r])
                    if j >= 2:
                        g_wait(ph, j - 2)
                        s_fire(ph, j - 2)
                    else:
                        @pl.when(g > 0)
                        def _(pph=pph, j=j):
                            g_wait(pph, j + 4)
                            s_fire(pph, j + 4)
                return carry

            lax.fori_loop(0, _NSUP, sup, 0)
            lastph = (_NSUP - 1) % 3
            g_wait(lastph, _SUB - 2)
            s_fire(lastph, _SUB - 2)
            g_wait(lastph, _SUB - 1)
            s_fire(lastph, _SUB - 1)
            for r in range(3):
                s_wait(lastph, r)
            plsc.subcore_barrier()
            pltpu.sync_copy(acc.at[pl.ds(s * _RS, _RS)],
                            out_hbm.at[pl.ds(base + s * _RS, _RS)])
            plsc.subcore_barrier()

    return prop


_deg_call = _make_deg()
_prop_call = _make_propagate()


def _dinv_body(degb, ob):
    i = pl.program_id(0)
    cnt = degb[0] + degb[1] + 1.0
    row = lax.broadcasted_iota(jnp.int32, (_BM, 1), 0) + i * _BM
    ob[...] = jnp.where(row < _N, lax.rsqrt(cnt), 0.0)


def _dinv(degp):
    return pl.pallas_call(
        _dinv_body,
        grid=(_NBLK,),
        in_specs=[pl.BlockSpec((2, _BM, 1), lambda i: (0, i, 0))],
        out_specs=pl.BlockSpec((_BM, 1), lambda i: (i, 0)),
        out_shape=jax.ShapeDtypeStruct((_NPAD, 1), jnp.float32),
    )(degp)


def _mm1_body(xb, wb, db, ob):
    ob[...] = jnp.dot(xb[...], wb[...],
                      preferred_element_type=jnp.float32) * db[...]


def _mm1(xp, W, dinv):
    return pl.pallas_call(
        _mm1_body,
        grid=(_NBLK,),
        in_specs=[
            pl.BlockSpec((_BM, _D), lambda i: (i, 0)),
            pl.BlockSpec((_D, _D), lambda i: (0, 0)),
            pl.BlockSpec((_BM, 1), lambda i: (i, 0)),
        ],
        out_specs=pl.BlockSpec((_BM, _D), lambda i: (i, 0)),
        out_shape=jax.ShapeDtypeStruct((_NPAD, _D), jnp.float32),
    )(xp, W, dinv)


def _mm2_body(pb, db, bb, wb, ob):
    h = jnp.maximum(pb[...] * db[...] + bb[...], 0.0)
    ob[...] = jnp.dot(h, wb[...], preferred_element_type=jnp.float32) * db[...]


def _mm2(p, dinv, brow, W):
    return pl.pallas_call(
        _mm2_body,
        grid=(_NBLK,),
        in_specs=[
            pl.BlockSpec((_BM, _D), lambda i: (i, 0)),
            pl.BlockSpec((_BM, 1), lambda i: (i, 0)),
            pl.BlockSpec((1, _D), lambda i: (0, 0)),
            pl.BlockSpec((_D, _D), lambda i: (0, 0)),
        ],
        out_specs=pl.BlockSpec((_BM, _D), lambda i: (i, 0)),
        out_shape=jax.ShapeDtypeStruct((_NPAD, _D), jnp.float32),
    )(p, dinv, brow, W)


def _pool_body(pb, db, bb, batchb, wlb, blb, ob, acc_s, acc_c):
    i = pl.program_id(0)

    @pl.when(i == 0)
    def _():
        acc_s[...] = jnp.zeros_like(acc_s)
        acc_c[...] = jnp.zeros_like(acc_c)

    h = pb[...] * db[...] + bb[...]
    gid = jnp.reshape(batchb[...], (1, _BM))
    oht = (lax.broadcasted_iota(jnp.int32, (_NG, _BM), 0) == gid
           ).astype(jnp.float32)
    acc_s[...] += jnp.dot(oht, h, preferred_element_type=jnp.float32)
    acc_c[...] += jnp.sum(oht, axis=1, keepdims=True)

    @pl.when(i == _NBLK - 1)
    def _():
        g = acc_s[...] / jnp.maximum(acc_c[...], 1.0)
        ob[...] = jnp.dot(g, wlb[...],
                          preferred_element_type=jnp.float32) + blb[...]


def _pool(p3, dinv, b3row, batch3, Wlp, blp):
    return pl.pallas_call(
        _pool_body,
        grid=(_NBLK,),
        in_specs=[
            pl.BlockSpec((_BM, _D), lambda i: (i, 0)),
            pl.BlockSpec((_BM, 1), lambda i: (i, 0)),
            pl.BlockSpec((1, _D), lambda i: (0, 0)),
            pl.BlockSpec((1, 1, _BM), lambda i: (i, 0, 0)),
            pl.BlockSpec((_D, 128), lambda i: (0, 0)),
            pl.BlockSpec((1, 128), lambda i: (0, 0)),
        ],
        out_specs=pl.BlockSpec((_NG, 128), lambda i: (0, 0)),
        out_shape=jax.ShapeDtypeStruct((_NG, 128), jnp.float32),
        scratch_shapes=[
            pltpu.VMEM((_NG, _D), jnp.float32),
            pltpu.VMEM((_NG, 1), jnp.float32),
        ],
    )(p3, dinv, b3row, batch3, Wlp, blp)


def kernel(x, edge_index, batch, W1, b1, W2, b2, W3, b3, Wl, bl):
    src = edge_index[0]
    dst = edge_index[1]
    srcp = jnp.concatenate(
        [src, jnp.zeros((_EPAD - _E,), jnp.int32)]).reshape(_EROWS, _CH)
    dstp = jnp.concatenate(
        [dst, jnp.full((_EPAD - _E,), _NPAD, jnp.int32)]).reshape(_EROWS, _CH)
    xp = jnp.pad(x, ((0, _NPAD - _N), (0, _D - x.shape[1])))
    W1p = jnp.pad(W1, ((0, _D - W1.shape[0]), (0, 0)))
    batch3 = jnp.pad(batch, (0, _NPAD - _N),
                     constant_values=_NG).reshape(_NBLK, 1, _BM)

    degp = _deg_call(dstp).reshape(2, _NPAD, 1)
    dinv = _dinv(degp)

    t = _mm1(xp, W1p, dinv)
    p = _prop_call(t, srcp, dstp)
    t = _mm2(p, dinv, b1.reshape(1, _D), W2)
    p = _prop_call(t, srcp, dstp)
    t = _mm2(p, dinv, b2.reshape(1, _D), W3)
    p = _prop_call(t, srcp, dstp)

    Wlp = jnp.pad(Wl, ((0, 0), (0, 128 - Wl.shape[1])))
    blp = jnp.pad(bl, (0, 128 - bl.shape[0])).reshape(1, 128)
    out = _pool(p, dinv, b3.reshape(1, _D), batch3, Wlp, blp)
    return out[:, :Wl.shape[1]]


# speedup vs baseline: 1.0621x; 1.0621x over previous
import functools

import jax
import jax.numpy as jnp
from jax import lax
from jax.experimental import pallas as pl
from jax.experimental.pallas import tpu as pltpu
from jax.experimental.pallas import tpu_sc as plsc

_N = 100000
_E = 1600000
_NG = 128
_D = 64
_NB = 4
_R = 25088
_NPAD = _NB * _R
_RS = _R // 16
_CH = 128
_SUB = 8
_ET = 100352
_NSUP = _ET // (_SUB * _CH)
_EPAD = 16 * _ET
_EROWS = _EPAD // _CH
_ETD = _EPAD // 32
_NCHD = _ETD // _CH
_DS = 6288
_BM = 512
_NBLK = _NPAD // _BM


def _make_deg():
    mesh = plsc.VectorSubcoreMesh(core_axis_name="c", subcore_axis_name="s")

    @functools.partial(
        pl.kernel,
        out_type=jax.ShapeDtypeStruct((2, _NPAD), jnp.float32),
        mesh=mesh,
        compiler_params=pltpu.CompilerParams(use_tc_tiling_on_sc=False),
        scratch_types=[
            pltpu.VMEM_SHARED((16 * _DS,), jnp.float32),
            pltpu.VMEM((_CH,), jnp.int32),
            pltpu.VMEM((_CH,), jnp.float32),
            pltpu.VMEM((_DS,), jnp.float32),
        ],
    )
    def deg(dstp_hbm, out_hbm, acc, didx, ones, zbuf):
        c = lax.axis_index("c")
        s = lax.axis_index("s")
        one_v = jnp.full((16,), 1.0, jnp.float32)
        zero_v = jnp.zeros((16,), jnp.float32)
        for j in range(_CH // 16):
            ones[pl.ds(j * 16, 16)] = one_v

        def zb(i, carry):
            zbuf[pl.ds(i * 16, 16)] = zero_v
            return carry

        lax.fori_loop(0, _DS // 16, zb, 0)
        pltpu.sync_copy(zbuf, acc.at[pl.ds(s * _DS, _DS)])
        plsc.subcore_barrier()

        base_r = (c * 16 + s) * _NCHD

        def chunk(g, carry):
            pltpu.sync_copy(dstp_hbm.at[base_r + g], didx)
            pltpu.sync_copy(ones, acc.at[didx], add=True)
            return carry

        lax.fori_loop(0, _NCHD, chunk, 0)
        plsc.subcore_barrier()
        pltpu.sync_copy(acc.at[pl.ds(s * (_NPAD // 16), _NPAD // 16)],
                        out_hbm.at[c, pl.ds(s * (_NPAD // 16), _NPAD // 16)])

    return deg


def _make_propagate():
    mesh = plsc.VectorSubcoreMesh(core_axis_name="c", subcore_axis_name="s")

    @functools.partial(
        pl.kernel,
        out_type=jax.ShapeDtypeStruct((_NPAD, _D), jnp.float32),
        mesh=mesh,
        compiler_params=pltpu.CompilerParams(use_tc_tiling_on_sc=False),
        scratch_types=[
            pltpu.VMEM_SHARED((_R + 8, _D), jnp.float32),
            pltpu.VMEM((2, _SUB, _CH), jnp.int32),
            pltpu.VMEM((2, _SUB, _CH), jnp.int32),
            pltpu.VMEM((2, _SUB, _CH), jnp.int32),
            pltpu.VMEM((2, _CH, _D), jnp.float32),
            pltpu.SemaphoreType.DMA((2,)),
            pltpu.SemaphoreType.DMA((2,)),
            pltpu.SemaphoreType.DMA((2,)),
        ],
    )
    def prop(t_hbm, srcp_hbm, dstp_hbm, out_hbm, acc, sidx, didx, lidx, rows,
             gsem, ssem, isem):
        c = lax.axis_index("c")
        s = lax.axis_index("s")
        tile_r0 = s * (_ET // _CH)

        def idx_load(g, ph):
            roff = tile_r0 + g * _SUB
            pltpu.async_copy(srcp_hbm.at[pl.ds(roff, _SUB)], sidx.at[ph],
                             isem.at[0])
            pltpu.async_copy(dstp_hbm.at[pl.ds(roff, _SUB)], didx.at[ph],
                             isem.at[1])

        def idx_wait(g, ph):
            roff = tile_r0 + g * _SUB
            pltpu.make_async_copy(srcp_hbm.at[pl.ds(roff, _SUB)], sidx.at[ph],
                                  isem.at[0]).wait()
            pltpu.make_async_copy(dstp_hbm.at[pl.ds(roff, _SUB)], didx.at[ph],
                                  isem.at[1]).wait()

        def s_wait(ph, j):
            pltpu.make_async_copy(rows.at[j & 1], acc.at[lidx.at[ph, j]],
                                  ssem.at[j & 1]).wait()

        for bb in range(2):
            b = 2 * c + bb
            base = b * _R
            pltpu.sync_copy(t_hbm.at[pl.ds(base + s * _RS, _RS)],
                            acc.at[pl.ds(s * _RS, _RS)])
            idx_load(0, 0)
            plsc.subcore_barrier()

            def sup(g, carry):
                ph = lax.rem(g, 2)
                idx_wait(g, ph)

                @pl.when(g + 1 < _NSUP)
                def _():
                    idx_load(g + 1, 1 - ph)

                for j in range(_SUB):
                    for k in range(_CH // 16):
                        dv = didx[ph, j, pl.ds(k * 16, 16)] - base
                        ok = (dv >= 0) & (dv < _R)
                        lidx[ph, j, pl.ds(k * 16, 16)] = jnp.where(ok, dv, _R)
                for j in range(_SUB):
                    if j < 2:
                        @pl.when(g > 0)
                        def _(ph=ph, j=j):
                            s_wait(ph, j)
                    else:
                        s_wait(ph, j)
                    pltpu.async_copy(t_hbm.at[sidx.at[ph, j]],
                                     rows.at[j & 1], gsem.at[j & 1])
                    if j > 0:
                        pltpu.make_async_copy(t_hbm.at[sidx.at[ph, j - 1]],
                                              rows.at[(j - 1) & 1],
                                              gsem.at[(j - 1) & 1]).wait()
                        pltpu.async_copy(rows.at[(j - 1) & 1],
                                         acc.at[lidx.at[ph, j - 1]],
                                         ssem.at[(j - 1) & 1], add=True)
                pltpu.make_async_copy(t_hbm.at[sidx.at[ph, _SUB - 1]],
                                      rows.at[(_SUB - 1) & 1],
                                      gsem.at[(_SUB - 1) & 1]).wait()
                pltpu.async_copy(rows.at[(_SUB - 1) & 1],
                                 acc.at[lidx.at[ph, _SUB - 1]],
                                 ssem.at[(_SUB - 1) & 1], add=True)
                return carry

            lax.fori_loop(0, _NSUP, sup, 0)
            lastph = (_NSUP - 1) & 1
            s_wait(lastph, _SUB - 2)
            s_wait(lastph, _SUB - 1)
            plsc.subcore_barrier()
            pltpu.sync_copy(acc.at[pl.ds(s * _RS, _RS)],
                            out_hbm.at[pl.ds(base + s * _RS, _RS)])
            plsc.subcore_barrier()

    return prop


_deg_call = _make_deg()
_prop_call = _make_propagate()


def _dinv_body(degb, ob):
    i = pl.program_id(0)
    cnt = degb[0] + degb[1] + 1.0
    row = lax.broadcasted_iota(jnp.int32, (_BM, 1), 0) + i * _BM
    ob[...] = jnp.where(row < _N, lax.rsqrt(cnt), 0.0)


def _dinv(degp):
    return pl.pallas_call(
        _dinv_body,
        grid=(_NBLK,),
        in_specs=[pl.BlockSpec((2, _BM, 1), lambda i: (0, i, 0))],
        out_specs=pl.BlockSpec((_BM, 1), lambda i: (i, 0)),
        out_shape=jax.ShapeDtypeStruct((_NPAD, 1), jnp.float32),
    )(degp)


def _mm1_body(xb, wb, db, ob):
    ob[...] = jnp.dot(xb[...], wb[...],
                      preferred_element_type=jnp.float32) * db[...]


def _mm1(xp, W, dinv):
    return pl.pallas_call(
        _mm1_body,
        grid=(_NBLK,),
        in_specs=[
            pl.BlockSpec((_BM, _D), lambda i: (i, 0)),
            pl.BlockSpec((_D, _D), lambda i: (0, 0)),
            pl.BlockSpec((_BM, 1), lambda i: (i, 0)),
        ],
        out_specs=pl.BlockSpec((_BM, _D), lambda i: (i, 0)),
        out_shape=jax.ShapeDtypeStruct((_NPAD, _D), jnp.float32),
    )(xp, W, dinv)


def _mm2_body(pb, db, bb, wb, ob):
    h = jnp.maximum(pb[...] * db[...] + bb[...], 0.0)
    ob[...] = jnp.dot(h, wb[...], preferred_element_type=jnp.float32) * db[...]


def _mm2(p, dinv, brow, W):
    return pl.pallas_call(
        _mm2_body,
        grid=(_NBLK,),
        in_specs=[
            pl.BlockSpec((_BM, _D), lambda i: (i, 0)),
            pl.BlockSpec((_BM, 1), lambda i: (i, 0)),
            pl.BlockSpec((1, _D), lambda i: (0, 0)),
            pl.BlockSpec((_D, _D), lambda i: (0, 0)),
        ],
        out_specs=pl.BlockSpec((_BM, _D), lambda i: (i, 0)),
        out_shape=jax.ShapeDtypeStruct((_NPAD, _D), jnp.float32),
    )(p, dinv, brow, W)


def _pool_body(pb, db, bb, batchb, wlb, blb, ob, acc_s, acc_c):
    i = pl.program_id(0)

    @pl.when(i == 0)
    def _():
        acc_s[...] = jnp.zeros_like(acc_s)
        acc_c[...] = jnp.zeros_like(acc_c)

    h = pb[...] * db[...] + bb[...]
    gid = jnp.reshape(batchb[...], (1, _BM))
    oht = (lax.broadcasted_iota(jnp.int32, (_NG, _BM), 0) == gid
           ).astype(jnp.float32)
    acc_s[...] += jnp.dot(oht, h, preferred_element_type=jnp.float32)
    acc_c[...] += jnp.sum(oht, axis=1, keepdims=True)

    @pl.when(i == _NBLK - 1)
    def _():
        g = acc_s[...] / jnp.maximum(acc_c[...], 1.0)
        ob[...] = jnp.dot(g, wlb[...],
                          preferred_element_type=jnp.float32) + blb[...]


def _pool(p3, dinv, b3row, batch3, Wlp, blp):
    return pl.pallas_call(
        _pool_body,
        grid=(_NBLK,),
        in_specs=[
            pl.BlockSpec((_BM, _D), lambda i: (i, 0)),
            pl.BlockSpec((_BM, 1), lambda i: (i, 0)),
            pl.BlockSpec((1, _D), lambda i: (0, 0)),
            pl.BlockSpec((1, 1, _BM), lambda i: (i, 0, 0)),
            pl.BlockSpec((_D, 128), lambda i: (0, 0)),
            pl.BlockSpec((1, 128), lambda i: (0, 0)),
        ],
        out_specs=pl.BlockSpec((_NG, 128), lambda i: (0, 0)),
        out_shape=jax.ShapeDtypeStruct((_NG, 128), jnp.float32),
        scratch_shapes=[
            pltpu.VMEM((_NG, _D), jnp.float32),
            pltpu.VMEM((_NG, 1), jnp.float32),
        ],
    )(p3, dinv, b3row, batch3, Wlp, blp)


def kernel(x, edge_index, batch, W1, b1, W2, b2, W3, b3, Wl, bl):
    src = edge_index[0]
    dst = edge_index[1]
    srcp = jnp.concatenate(
        [src, jnp.zeros((_EPAD - _E,), jnp.int32)]).reshape(_EROWS, _CH)
    dstp = jnp.concatenate(
        [dst, jnp.full((_EPAD - _E,), _NPAD, jnp.int32)]).reshape(_EROWS, _CH)
    xp = jnp.pad(x, ((0, _NPAD - _N), (0, _D - x.shape[1])))
    W1p = jnp.pad(W1, ((0, _D - W1.shape[0]), (0, 0)))
    batch3 = jnp.pad(batch, (0, _NPAD - _N),
                     constant_values=_NG).reshape(_NBLK, 1, _BM)

    degp = _deg_call(dstp).reshape(2, _NPAD, 1)
    dinv = _dinv(degp)

    t = _mm1(xp, W1p, dinv)
    p = _prop_call(t, srcp, dstp)
    t = _mm2(p, dinv, b1.reshape(1, _D), W2)
    p = _prop_call(t, srcp, dstp)
    t = _mm2(p, dinv, b2.reshape(1, _D), W3)
    p = _prop_call(t, srcp, dstp)

    Wlp = jnp.pad(Wl, ((0, 0), (0, 128 - Wl.shape[1])))
    blp = jnp.pad(bl, (0, 128 - bl.shape[0])).reshape(1, 128)
    out = _pool(p, dinv, b3.reshape(1, _D), batch3, Wlp, blp)
    return out[:, :Wl.shape[1]]
